# baseline (speedup 1.0000x reference)
import jax
import jax.numpy as jnp
from jax.experimental import pallas as pl

BATCH = 1024
IN_FEATURES = 128
OUT_FEATURES = 128
GRID_SIZE = 8
MIN_VALUE = -2.0
MAX_VALUE = 2.0


def _kan_kernel(x_ref, basis_ref, bias_ref, out_ref):
    x = x_ref[:]
    scaled = (jnp.clip(x, MIN_VALUE, MAX_VALUE) - MIN_VALUE) * (
        (GRID_SIZE - 1) / (MAX_VALUE - MIN_VALUE)
    )
    acc = jnp.broadcast_to(bias_ref[:], out_ref.shape)
    for g in range(GRID_SIZE):
        w = jnp.maximum(1.0 - jnp.abs(scaled - float(g)), 0.0).astype(jnp.bfloat16)
        bg = basis_ref[:, g * OUT_FEATURES:(g + 1) * OUT_FEATURES]
        acc = acc + jnp.dot(w, bg, preferred_element_type=jnp.float32)
    out_ref[:] = acc


def kernel(inputs, basis, bias):
    basis_t = jnp.transpose(basis, (1, 2, 0)).astype(jnp.bfloat16).reshape(
        IN_FEATURES, GRID_SIZE * OUT_FEATURES)
    bias2d = bias.reshape(1, OUT_FEATURES)
    return pl.pallas_call(
        _kan_kernel,
        grid=(1,),
        in_specs=[
            pl.BlockSpec((BATCH, IN_FEATURES), lambda i: (0, 0)),
            pl.BlockSpec((IN_FEATURES, GRID_SIZE * OUT_FEATURES), lambda i: (0, 0)),
            pl.BlockSpec((1, OUT_FEATURES), lambda i: (0, 0)),
        ],
        out_specs=pl.BlockSpec((BATCH, OUT_FEATURES), lambda i: (0, 0)),
        out_shape=jax.ShapeDtypeStruct((BATCH, OUT_FEATURES), jnp.float32),
    )(inputs, basis_t, bias2d)
